# baseline (speedup 1.0000x reference)
import functools

import jax
import jax.numpy as jnp
from jax import lax
from jax.experimental import pallas as pl
from jax.experimental.pallas import tpu as pltpu
from jax.experimental.pallas import tpu_sc as plsc

B, L, EMB, HID, K, MEL = 16, 512, 256, 256, 3, 4096

NC, NS = 2, 16
NW = NC * NS
ROWS = B * MEL
ROWS_PER_W = ROWS // NW
CH = 128
NCHUNK = ROWS_PER_W // CH
PAD_ROWS = 8
ZERO_ROW = B * L


def _ln_relu(h, g, be):
    m = jnp.mean(h, axis=-1, keepdims=True)
    r = h - m
    v = jnp.mean(r * r, axis=-1, keepdims=True)
    return jax.nn.relu(r * jax.lax.rsqrt(v + 1e-5) * g + be)


def _conv3(h, wt, b):
    z = jnp.zeros((1, h.shape[1]), h.dtype)
    prev = jnp.concatenate([z, h[:-1]], axis=0)
    nxt = jnp.concatenate([h[1:], z], axis=0)
    y = (jnp.dot(prev, wt[0], preferred_element_type=jnp.float32)
         + jnp.dot(h, wt[1], preferred_element_type=jnp.float32)
         + jnp.dot(nxt, wt[2], preferred_element_type=jnp.float32))
    return y + b


def _tc_body(x_ref, t_ref, w1_ref, b1_ref, g1_ref, be1_ref,
             w2_ref, b2_ref, g2_ref, be2_ref, wl_ref, bl_ref,
             dur_ref, gidx_ref):
    b = pl.program_id(0)
    xb = x_ref[0]

    h = _ln_relu(_conv3(xb, w1_ref[...], b1_ref[...]), g1_ref[...], be1_ref[...])
    h = _ln_relu(_conv3(h, w2_ref[...], b2_ref[...]), g2_ref[...], be2_ref[...])
    dur = jax.nn.relu(jnp.dot(h, wl_ref[...], preferred_element_type=jnp.float32)
                      + bl_ref[...])
    dur_ref[0] = dur

    tgt = t_ref[0].astype(jnp.float32)
    row = lax.broadcasted_iota(jnp.int32, (L, L), 0)
    col = lax.broadcasted_iota(jnp.int32, (L, L), 1)
    lt = (col <= row).astype(jnp.float32)
    cum = jnp.dot(lt, tgt, preferred_element_type=jnp.float32)

    tt = lax.broadcasted_iota(jnp.int32, (1, MEL), 1).astype(jnp.float32)
    le = (cum <= tt).astype(jnp.float32)
    idx = jnp.sum(le, axis=0, keepdims=True).astype(jnp.int32)
    gidx = jnp.where(idx < L, b * L + idx, ZERO_ROW)
    gidx_ref[0] = gidx


def _tc_predict(x, t_col, w1t, b1r, g1r, be1r, w2t, b2r, g2r, be2r, wl, blr):
    full2 = lambda a: pl.BlockSpec(a, lambda b: (0, 0))
    full3 = lambda a: pl.BlockSpec(a, lambda b: (0, 0, 0))
    return pl.pallas_call(
        _tc_body,
        grid=(B,),
        in_specs=[
            pl.BlockSpec((1, L, EMB), lambda b: (b, 0, 0)),
            pl.BlockSpec((1, L, 1), lambda b: (b, 0, 0)),
            full3((K, EMB, HID)), full2((1, HID)), full2((1, HID)), full2((1, HID)),
            full3((K, HID, HID)), full2((1, HID)), full2((1, HID)), full2((1, HID)),
            full2((HID, 1)), full2((1, 1)),
        ],
        out_specs=[
            pl.BlockSpec((1, L, 1), lambda b: (b, 0, 0)),
            pl.BlockSpec((1, 1, MEL), lambda b: (b, 0, 0)),
        ],
        out_shape=[
            jax.ShapeDtypeStruct((B, L, 1), jnp.float32),
            jax.ShapeDtypeStruct((B, 1, MEL), jnp.int32),
        ],
    )(x, t_col, w1t, b1r, g1r, be1r, w2t, b2r, g2r, be2r, wl, blr)


NBUF = 3


def _sc_expand_body(table_hbm, gidx_hbm, out_hbm, idx_all, *bufs_and_sems):
    rows_v = bufs_and_sems[:NBUF]
    gsems = bufs_and_sems[NBUF:2 * NBUF]
    ssems = bufs_and_sems[2 * NBUF:3 * NBUF]

    wid = lax.axis_index("s") * NC + lax.axis_index("c")
    base = pl.multiple_of(wid * ROWS_PER_W, ROWS_PER_W)

    pltpu.sync_copy(gidx_hbm.at[wid], idx_all)

    gath = [None] * NBUF
    scat = [None] * NBUF
    for c in range(min(NBUF, NCHUNK)):
        gath[c] = pltpu.async_copy(table_hbm.at[idx_all.at[c]], rows_v[c],
                                   gsems[c])
    for c in range(NCHUNK):
        bi = c % NBUF
        gath[bi].wait()
        scat[bi] = pltpu.async_copy(
            rows_v[bi], out_hbm.at[pl.ds(base + c * CH, CH)], ssems[bi])
        nc = c + NBUF
        if nc < NCHUNK:
            scat[bi].wait()
            gath[bi] = pltpu.async_copy(table_hbm.at[idx_all.at[nc]],
                                        rows_v[bi], gsems[bi])
    for c in range(max(NCHUNK - NBUF, 0), NCHUNK):
        scat[c % NBUF].wait()


@functools.cache
def _sc_expand():
    return pl.kernel(
        _sc_expand_body,
        out_type=jax.ShapeDtypeStruct((ROWS, EMB), jnp.float32),
        mesh=plsc.VectorSubcoreMesh(core_axis_name="c", subcore_axis_name="s",
                                    num_cores=NC, num_subcores=NS),
        scratch_types=[
            pltpu.VMEM((NCHUNK, CH), jnp.int32),
            *[pltpu.VMEM((CH, EMB), jnp.float32) for _ in range(NBUF)],
            *[pltpu.SemaphoreType.DMA for _ in range(2 * NBUF)],
        ],
    )


def kernel(x, target, mel_max_len, W1, b1, g1, be1, W2, b2, g2, be2, Wl, bl):
    w1t = jnp.transpose(W1, (2, 1, 0))
    w2t = jnp.transpose(W2, (2, 1, 0))
    t_col = target.reshape(B, L, 1)
    row2 = lambda a: a.reshape(1, -1)

    dur, gidx3 = _tc_predict(
        x, t_col, w1t, row2(b1), row2(g1), row2(be1),
        w2t, row2(b2), row2(g2), row2(be2), Wl, bl.reshape(1, 1))

    table = jnp.concatenate(
        [x.reshape(B * L, EMB), jnp.zeros((PAD_ROWS, EMB), x.dtype)], axis=0)
    out_flat = _sc_expand()(table, gidx3.reshape(NW, NCHUNK, CH))
    output = out_flat.reshape(B, MEL, EMB)
    return (output, dur)


# speedup vs baseline: 11.3702x; 11.3702x over previous
import functools

import jax
import jax.numpy as jnp
from jax import lax
from jax.experimental import pallas as pl
from jax.experimental.pallas import tpu as pltpu
from jax.experimental.pallas import tpu_sc as plsc

B, L, EMB, HID, K, MEL = 16, 512, 256, 256, 3, 4096

NC, NS = 2, 16
NW = NC * NS
ROWS = B * MEL
ROWS_PER_W = ROWS // NW
CH = 128
NCHUNK = ROWS_PER_W // CH
PAD_ROWS = 1024
ZERO_ROW = B * L


def _ln_relu(h, g, be):
    m = jnp.mean(h, axis=-1, keepdims=True)
    r = h - m
    v = jnp.mean(r * r, axis=-1, keepdims=True)
    return jax.nn.relu(r * jax.lax.rsqrt(v + 1e-5) * g + be)


def _conv3(h, wt, b):
    z = jnp.zeros((1, h.shape[1]), h.dtype)
    prev = jnp.concatenate([z, h[:-1]], axis=0)
    nxt = jnp.concatenate([h[1:], z], axis=0)
    y = (jnp.dot(prev, wt[0], preferred_element_type=jnp.float32)
         + jnp.dot(h, wt[1], preferred_element_type=jnp.float32)
         + jnp.dot(nxt, wt[2], preferred_element_type=jnp.float32))
    return y + b


def _tc_body(x_ref, t_ref, w1_ref, b1_ref, g1_ref, be1_ref,
             w2_ref, b2_ref, g2_ref, be2_ref, wl_ref, bl_ref,
             dur_ref, gidx_ref):
    b = pl.program_id(0)
    xb = x_ref[0]

    h = _ln_relu(_conv3(xb, w1_ref[...], b1_ref[...]), g1_ref[...], be1_ref[...])
    h = _ln_relu(_conv3(h, w2_ref[...], b2_ref[...]), g2_ref[...], be2_ref[...])
    dur = jax.nn.relu(jnp.dot(h, wl_ref[...], preferred_element_type=jnp.float32)
                      + bl_ref[...])
    dur_ref[0] = dur

    tgt = t_ref[0].astype(jnp.float32)
    row = lax.broadcasted_iota(jnp.int32, (L, L), 0)
    col = lax.broadcasted_iota(jnp.int32, (L, L), 1)
    lt = (col <= row).astype(jnp.float32)
    cum = jnp.dot(lt, tgt, preferred_element_type=jnp.float32)

    tt = lax.broadcasted_iota(jnp.int32, (1, MEL), 1).astype(jnp.float32)
    le = (cum <= tt).astype(jnp.float32)
    idx = jnp.sum(le, axis=0, keepdims=True).astype(jnp.int32)
    ti = lax.broadcasted_iota(jnp.int32, (1, MEL), 1)
    gidx = jnp.where(idx < L, b * L + idx,
                     ZERO_ROW + (ti & (PAD_ROWS - 1)))
    gidx_ref[0] = gidx


def _tc_predict(x, t_col, w1t, b1r, g1r, be1r, w2t, b2r, g2r, be2r, wl, blr):
    full2 = lambda a: pl.BlockSpec(a, lambda b: (0, 0))
    full3 = lambda a: pl.BlockSpec(a, lambda b: (0, 0, 0))
    return pl.pallas_call(
        _tc_body,
        grid=(B,),
        in_specs=[
            pl.BlockSpec((1, L, EMB), lambda b: (b, 0, 0)),
            pl.BlockSpec((1, L, 1), lambda b: (b, 0, 0)),
            full3((K, EMB, HID)), full2((1, HID)), full2((1, HID)), full2((1, HID)),
            full3((K, HID, HID)), full2((1, HID)), full2((1, HID)), full2((1, HID)),
            full2((HID, 1)), full2((1, 1)),
        ],
        out_specs=[
            pl.BlockSpec((1, L, 1), lambda b: (b, 0, 0)),
            pl.BlockSpec((1, 1, MEL), lambda b: (b, 0, 0)),
        ],
        out_shape=[
            jax.ShapeDtypeStruct((B, L, 1), jnp.float32),
            jax.ShapeDtypeStruct((B, 1, MEL), jnp.int32),
        ],
    )(x, t_col, w1t, b1r, g1r, be1r, w2t, b2r, g2r, be2r, wl, blr)


NBUF = 3


def _sc_expand_body(table_hbm, gidx_hbm, out_hbm, idx_all, *bufs_and_sems):
    rows_v = bufs_and_sems[:NBUF]
    gsems = bufs_and_sems[NBUF:2 * NBUF]
    ssems = bufs_and_sems[2 * NBUF:3 * NBUF]

    wid = lax.axis_index("s") * NC + lax.axis_index("c")
    base = pl.multiple_of(wid * ROWS_PER_W, ROWS_PER_W)

    pltpu.sync_copy(gidx_hbm.at[wid], idx_all)

    gath = [None] * NBUF
    scat = [None] * NBUF
    for c in range(min(NBUF, NCHUNK)):
        gath[c] = pltpu.async_copy(table_hbm.at[idx_all.at[c]], rows_v[c],
                                   gsems[c])
    for c in range(NCHUNK):
        bi = c % NBUF
        gath[bi].wait()
        scat[bi] = pltpu.async_copy(
            rows_v[bi], out_hbm.at[pl.ds(base + c * CH, CH)], ssems[bi])
        nc = c + NBUF
        if nc < NCHUNK:
            scat[bi].wait()
            gath[bi] = pltpu.async_copy(table_hbm.at[idx_all.at[nc]],
                                        rows_v[bi], gsems[bi])
    for c in range(max(NCHUNK - NBUF, 0), NCHUNK):
        scat[c % NBUF].wait()


@functools.cache
def _sc_expand():
    return pl.kernel(
        _sc_expand_body,
        out_type=jax.ShapeDtypeStruct((ROWS, EMB), jnp.float32),
        mesh=plsc.VectorSubcoreMesh(core_axis_name="c", subcore_axis_name="s",
                                    num_cores=NC, num_subcores=NS),
        scratch_types=[
            pltpu.VMEM((NCHUNK, CH), jnp.int32),
            *[pltpu.VMEM((CH, EMB), jnp.float32) for _ in range(NBUF)],
            *[pltpu.SemaphoreType.DMA for _ in range(2 * NBUF)],
        ],
    )


def kernel(x, target, mel_max_len, W1, b1, g1, be1, W2, b2, g2, be2, Wl, bl):
    w1t = jnp.transpose(W1, (2, 1, 0))
    w2t = jnp.transpose(W2, (2, 1, 0))
    t_col = target.reshape(B, L, 1)
    row2 = lambda a: a.reshape(1, -1)

    dur, gidx3 = _tc_predict(
        x, t_col, w1t, row2(b1), row2(g1), row2(be1),
        w2t, row2(b2), row2(g2), row2(be2), Wl, bl.reshape(1, 1))

    table = jnp.concatenate(
        [x.reshape(B * L, EMB), jnp.zeros((PAD_ROWS, EMB), x.dtype)], axis=0)
    out_flat = _sc_expand()(table, gidx3.reshape(NW, NCHUNK, CH))
    output = out_flat.reshape(B, MEL, EMB)
    return (output, dur)


# speedup vs baseline: 13.5086x; 1.1881x over previous
import functools

import jax
import jax.numpy as jnp
from jax import lax
from jax.experimental import pallas as pl
from jax.experimental.pallas import tpu as pltpu
from jax.experimental.pallas import tpu_sc as plsc

B, L, EMB, HID, K, MEL = 16, 512, 256, 256, 3, 4096

NC, NS = 2, 16
NW = NC * NS
ROWS = B * MEL
ROWS_PER_W = ROWS // NW
CH = 128
NCHUNK = ROWS_PER_W // CH
PAD_ROWS = 1024
ZERO_ROW = B * L


def _ln_relu(h, g, be):
    m = jnp.mean(h, axis=-1, keepdims=True)
    r = h - m
    v = jnp.mean(r * r, axis=-1, keepdims=True)
    return jax.nn.relu(r * jax.lax.rsqrt(v + 1e-5) * g + be)


def _conv3(h, wt, b):
    z = jnp.zeros((1, h.shape[1]), h.dtype)
    prev = jnp.concatenate([z, h[:-1]], axis=0)
    nxt = jnp.concatenate([h[1:], z], axis=0)
    y = (jnp.dot(prev, wt[0], preferred_element_type=jnp.float32)
         + jnp.dot(h, wt[1], preferred_element_type=jnp.float32)
         + jnp.dot(nxt, wt[2], preferred_element_type=jnp.float32))
    return y + b


def _tc_pred_body(x_ref, w1_ref, b1_ref, g1_ref, be1_ref,
                  w2_ref, b2_ref, g2_ref, be2_ref, wl_ref, bl_ref, dur_ref):
    xb = x_ref[0]
    h = _ln_relu(_conv3(xb, w1_ref[...], b1_ref[...]), g1_ref[...], be1_ref[...])
    h = _ln_relu(_conv3(h, w2_ref[...], b2_ref[...]), g2_ref[...], be2_ref[...])
    dur = jax.nn.relu(jnp.dot(h, wl_ref[...], preferred_element_type=jnp.float32)
                      + bl_ref[...])
    dur_ref[0] = dur


def _tc_idx_body(t_ref, gidx_ref):
    b = pl.program_id(0)
    tgt = t_ref[0].astype(jnp.float32)
    row = lax.broadcasted_iota(jnp.int32, (L, L), 0)
    col = lax.broadcasted_iota(jnp.int32, (L, L), 1)
    lt = (col <= row).astype(jnp.float32)
    cum = jnp.dot(lt, tgt, preferred_element_type=jnp.float32)

    tt = lax.broadcasted_iota(jnp.int32, (1, MEL), 1).astype(jnp.float32)
    le = (cum <= tt).astype(jnp.float32)
    idx = jnp.sum(le, axis=0, keepdims=True).astype(jnp.int32)
    ti = lax.broadcasted_iota(jnp.int32, (1, MEL), 1)
    gidx = jnp.where(idx < L, b * L + idx,
                     ZERO_ROW + (ti & (PAD_ROWS - 1)))
    gidx_ref[0] = gidx


def _tc_predict(x, w1t, b1r, g1r, be1r, w2t, b2r, g2r, be2r, wl, blr):
    full2 = lambda a: pl.BlockSpec(a, lambda b: (0, 0))
    full3 = lambda a: pl.BlockSpec(a, lambda b: (0, 0, 0))
    return pl.pallas_call(
        _tc_pred_body,
        grid=(B,),
        in_specs=[
            pl.BlockSpec((1, L, EMB), lambda b: (b, 0, 0)),
            full3((K, EMB, HID)), full2((1, HID)), full2((1, HID)), full2((1, HID)),
            full3((K, HID, HID)), full2((1, HID)), full2((1, HID)), full2((1, HID)),
            full2((HID, 1)), full2((1, 1)),
        ],
        out_specs=pl.BlockSpec((1, L, 1), lambda b: (b, 0, 0)),
        out_shape=jax.ShapeDtypeStruct((B, L, 1), jnp.float32),
    )(x, w1t, b1r, g1r, be1r, w2t, b2r, g2r, be2r, wl, blr)


def _tc_idx(t_col):
    return pl.pallas_call(
        _tc_idx_body,
        grid=(B,),
        in_specs=[pl.BlockSpec((1, L, 1), lambda b: (b, 0, 0))],
        out_specs=pl.BlockSpec((1, 1, MEL), lambda b: (b, 0, 0)),
        out_shape=jax.ShapeDtypeStruct((B, 1, MEL), jnp.int32),
    )(t_col)


NBUF = 3


def _sc_expand_body(table_hbm, gidx_hbm, out_hbm, idx_all, *bufs_and_sems):
    rows_v = bufs_and_sems[:NBUF]
    gsems = bufs_and_sems[NBUF:2 * NBUF]
    ssems = bufs_and_sems[2 * NBUF:3 * NBUF]

    wid = lax.axis_index("s") * NC + lax.axis_index("c")
    base = pl.multiple_of(wid * ROWS_PER_W, ROWS_PER_W)

    pltpu.sync_copy(gidx_hbm.at[wid], idx_all)

    gath = [None] * NBUF
    scat = [None] * NBUF
    for c in range(min(NBUF, NCHUNK)):
        gath[c] = pltpu.async_copy(table_hbm.at[idx_all.at[c]], rows_v[c],
                                   gsems[c])
    for c in range(NCHUNK):
        bi = c % NBUF
        gath[bi].wait()
        scat[bi] = pltpu.async_copy(
            rows_v[bi], out_hbm.at[pl.ds(base + c * CH, CH)], ssems[bi])
        nc = c + NBUF
        if nc < NCHUNK:
            scat[bi].wait()
            gath[bi] = pltpu.async_copy(table_hbm.at[idx_all.at[nc]],
                                        rows_v[bi], gsems[bi])
    for c in range(max(NCHUNK - NBUF, 0), NCHUNK):
        scat[c % NBUF].wait()


@functools.cache
def _sc_expand():
    return pl.kernel(
        _sc_expand_body,
        out_type=jax.ShapeDtypeStruct((ROWS, EMB), jnp.float32),
        mesh=plsc.VectorSubcoreMesh(core_axis_name="c", subcore_axis_name="s",
                                    num_cores=NC, num_subcores=NS),
        scratch_types=[
            pltpu.VMEM((NCHUNK, CH), jnp.int32),
            *[pltpu.VMEM((CH, EMB), jnp.float32) for _ in range(NBUF)],
            *[pltpu.SemaphoreType.DMA for _ in range(2 * NBUF)],
        ],
    )


def kernel(x, target, mel_max_len, W1, b1, g1, be1, W2, b2, g2, be2, Wl, bl):
    w1t = jnp.transpose(W1, (2, 1, 0))
    w2t = jnp.transpose(W2, (2, 1, 0))
    t_col = target.reshape(B, L, 1)
    row2 = lambda a: a.reshape(1, -1)

    gidx3 = _tc_idx(t_col)
    table = jnp.concatenate(
        [x.reshape(B * L, EMB), jnp.zeros((PAD_ROWS, EMB), x.dtype)], axis=0)
    out_flat = _sc_expand()(table, gidx3.reshape(NW, NCHUNK, CH))
    dur = _tc_predict(
        x, w1t, row2(b1), row2(g1), row2(be1),
        w2t, row2(b2), row2(g2), row2(be2), Wl, bl.reshape(1, 1))
    output = out_flat.reshape(B, MEL, EMB)
    return (output, dur)


# speedup vs baseline: 13.6384x; 1.0096x over previous
import functools

import jax
import jax.numpy as jnp
from jax import lax
from jax.experimental import pallas as pl
from jax.experimental.pallas import tpu as pltpu
from jax.experimental.pallas import tpu_sc as plsc

B, L, EMB, HID, K, MEL = 16, 512, 256, 256, 3, 4096

NC, NS = 2, 16
NW = NC * NS
ROWS = B * MEL
ROWS_PER_W = ROWS // NW
CH = 128
NCHUNK = ROWS_PER_W // CH
PAD_ROWS = 1024
ZERO_ROW = B * L


def _ln_relu(h, g, be):
    m = jnp.mean(h, axis=-1, keepdims=True)
    r = h - m
    v = jnp.mean(r * r, axis=-1, keepdims=True)
    return jax.nn.relu(r * jax.lax.rsqrt(v + 1e-5) * g + be)


def _conv3(h, wt, b):
    z = jnp.zeros((1, h.shape[1]), h.dtype)
    prev = jnp.concatenate([z, h[:-1]], axis=0)
    nxt = jnp.concatenate([h[1:], z], axis=0)
    y = (jnp.dot(prev, wt[0], preferred_element_type=jnp.float32)
         + jnp.dot(h, wt[1], preferred_element_type=jnp.float32)
         + jnp.dot(nxt, wt[2], preferred_element_type=jnp.float32))
    return y + b


def _tc_pred_body(x_ref, w1_ref, b1_ref, g1_ref, be1_ref,
                  w2_ref, b2_ref, g2_ref, be2_ref, wl_ref, bl_ref, dur_ref):
    xb = x_ref[0]
    h = _ln_relu(_conv3(xb, w1_ref[...], b1_ref[...]), g1_ref[...], be1_ref[...])
    h = _ln_relu(_conv3(h, w2_ref[...], b2_ref[...]), g2_ref[...], be2_ref[...])
    dur = jax.nn.relu(jnp.dot(h, wl_ref[...], preferred_element_type=jnp.float32)
                      + bl_ref[...])
    dur_ref[0] = dur


def _tc_idx_body(t_ref, gidx_ref):
    b = pl.program_id(0)
    tgt = t_ref[0].astype(jnp.float32)
    row = lax.broadcasted_iota(jnp.int32, (L, L), 0)
    col = lax.broadcasted_iota(jnp.int32, (L, L), 1)
    lt = (col <= row).astype(jnp.float32)
    cum = jnp.dot(lt, tgt, preferred_element_type=jnp.float32)

    tt = lax.broadcasted_iota(jnp.int32, (1, MEL), 1).astype(jnp.float32)
    le = (cum <= tt).astype(jnp.bfloat16)
    ones = jnp.ones((1, L), jnp.bfloat16)
    idx = jnp.dot(ones, le,
                  preferred_element_type=jnp.float32).astype(jnp.int32)
    ti = lax.broadcasted_iota(jnp.int32, (1, MEL), 1)
    gidx = jnp.where(idx < L, b * L + idx,
                     ZERO_ROW + (ti & (PAD_ROWS - 1)))
    gidx_ref[0] = gidx


def _tc_predict(x, w1t, b1r, g1r, be1r, w2t, b2r, g2r, be2r, wl, blr):
    full2 = lambda a: pl.BlockSpec(a, lambda b: (0, 0))
    full3 = lambda a: pl.BlockSpec(a, lambda b: (0, 0, 0))
    return pl.pallas_call(
        _tc_pred_body,
        grid=(B,),
        in_specs=[
            pl.BlockSpec((1, L, EMB), lambda b: (b, 0, 0)),
            full3((K, EMB, HID)), full2((1, HID)), full2((1, HID)), full2((1, HID)),
            full3((K, HID, HID)), full2((1, HID)), full2((1, HID)), full2((1, HID)),
            full2((HID, 1)), full2((1, 1)),
        ],
        out_specs=pl.BlockSpec((1, L, 1), lambda b: (b, 0, 0)),
        out_shape=jax.ShapeDtypeStruct((B, L, 1), jnp.float32),
    )(x, w1t, b1r, g1r, be1r, w2t, b2r, g2r, be2r, wl, blr)


def _tc_idx(t_col):
    return pl.pallas_call(
        _tc_idx_body,
        grid=(B,),
        in_specs=[pl.BlockSpec((1, L, 1), lambda b: (b, 0, 0))],
        out_specs=pl.BlockSpec((1, 1, MEL), lambda b: (b, 0, 0)),
        out_shape=jax.ShapeDtypeStruct((B, 1, MEL), jnp.int32),
    )(t_col)


NBUF = 3


def _sc_expand_body(table_hbm, gidx_hbm, out_hbm, idx_all, *bufs_and_sems):
    rows_v = bufs_and_sems[:NBUF]
    gsems = bufs_and_sems[NBUF:2 * NBUF]
    ssems = bufs_and_sems[2 * NBUF:3 * NBUF]

    wid = lax.axis_index("s") * NC + lax.axis_index("c")
    base = pl.multiple_of(wid * ROWS_PER_W, ROWS_PER_W)

    pltpu.sync_copy(gidx_hbm.at[wid], idx_all)

    gath = [None] * NBUF
    scat = [None] * NBUF
    for c in range(min(NBUF, NCHUNK)):
        gath[c] = pltpu.async_copy(table_hbm.at[idx_all.at[c]], rows_v[c],
                                   gsems[c])
    for c in range(NCHUNK):
        bi = c % NBUF
        gath[bi].wait()
        scat[bi] = pltpu.async_copy(
            rows_v[bi], out_hbm.at[pl.ds(base + c * CH, CH)], ssems[bi])
        nc = c + NBUF
        if nc < NCHUNK:
            scat[bi].wait()
            gath[bi] = pltpu.async_copy(table_hbm.at[idx_all.at[nc]],
                                        rows_v[bi], gsems[bi])
    for c in range(max(NCHUNK - NBUF, 0), NCHUNK):
        scat[c % NBUF].wait()


@functools.cache
def _sc_expand():
    return pl.kernel(
        _sc_expand_body,
        out_type=jax.ShapeDtypeStruct((ROWS, EMB), jnp.float32),
        mesh=plsc.VectorSubcoreMesh(core_axis_name="c", subcore_axis_name="s",
                                    num_cores=NC, num_subcores=NS),
        scratch_types=[
            pltpu.VMEM((NCHUNK, CH), jnp.int32),
            *[pltpu.VMEM((CH, EMB), jnp.float32) for _ in range(NBUF)],
            *[pltpu.SemaphoreType.DMA for _ in range(2 * NBUF)],
        ],
    )


def kernel(x, target, mel_max_len, W1, b1, g1, be1, W2, b2, g2, be2, Wl, bl):
    w1t = jnp.transpose(W1, (2, 1, 0))
    w2t = jnp.transpose(W2, (2, 1, 0))
    t_col = target.reshape(B, L, 1)
    row2 = lambda a: a.reshape(1, -1)

    gidx3 = _tc_idx(t_col)
    table = jnp.concatenate(
        [x.reshape(B * L, EMB), jnp.zeros((PAD_ROWS, EMB), x.dtype)], axis=0)
    out_flat = _sc_expand()(table, gidx3.reshape(NW, NCHUNK, CH))
    dur = _tc_predict(
        x, w1t, row2(b1), row2(g1), row2(be1),
        w2t, row2(b2), row2(g2), row2(be2), Wl, bl.reshape(1, 1))
    output = out_flat.reshape(B, MEL, EMB)
    return (output, dur)


# speedup vs baseline: 14.7785x; 1.0836x over previous
import functools

import jax
import jax.numpy as jnp
from jax import lax
from jax.experimental import pallas as pl
from jax.experimental.pallas import tpu as pltpu
from jax.experimental.pallas import tpu_sc as plsc

B, L, EMB, HID, K, MEL = 16, 512, 256, 256, 3, 4096

NC, NS = 2, 16
NW = NC * NS
ROWS = B * MEL
ROWS_PER_W = ROWS // NW
CH = 128
NCHUNK = ROWS_PER_W // CH
FLAG = 1 << 30


def _ln_relu(h, g, be):
    m = jnp.mean(h, axis=-1, keepdims=True)
    r = h - m
    v = jnp.mean(r * r, axis=-1, keepdims=True)
    return jax.nn.relu(r * jax.lax.rsqrt(v + 1e-5) * g + be)


def _conv3(h, wt, b):
    z = jnp.zeros((1, h.shape[1]), h.dtype)
    prev = jnp.concatenate([z, h[:-1]], axis=0)
    nxt = jnp.concatenate([h[1:], z], axis=0)
    y = (jnp.dot(prev, wt[0], preferred_element_type=jnp.float32)
         + jnp.dot(h, wt[1], preferred_element_type=jnp.float32)
         + jnp.dot(nxt, wt[2], preferred_element_type=jnp.float32))
    return y + b


def _tc_pred_body(x_ref, w1_ref, b1_ref, g1_ref, be1_ref,
                  w2_ref, b2_ref, g2_ref, be2_ref, wl_ref, bl_ref, dur_ref):
    xb = x_ref[0]
    h = _ln_relu(_conv3(xb, w1_ref[...], b1_ref[...]), g1_ref[...], be1_ref[...])
    h = _ln_relu(_conv3(h, w2_ref[...], b2_ref[...]), g2_ref[...], be2_ref[...])
    dur = jax.nn.relu(jnp.dot(h, wl_ref[...], preferred_element_type=jnp.float32)
                      + bl_ref[...])
    dur_ref[0] = dur


def _tc_idx_body(t_ref, gidx_ref, nreal_ref):
    b = pl.program_id(0)
    tgt = t_ref[0].astype(jnp.float32)
    row = lax.broadcasted_iota(jnp.int32, (L, L), 0)
    col = lax.broadcasted_iota(jnp.int32, (L, L), 1)
    lt = (col <= row).astype(jnp.float32)
    cum = jnp.dot(lt, tgt, preferred_element_type=jnp.float32)

    tt = lax.broadcasted_iota(jnp.int32, (1, MEL), 1).astype(jnp.float32)
    le = (cum <= tt).astype(jnp.bfloat16)
    ones = jnp.ones((1, L), jnp.bfloat16)
    idx = jnp.dot(ones, le,
                  preferred_element_type=jnp.float32).astype(jnp.int32)
    ti = lax.broadcasted_iota(jnp.int32, (1, MEL), 1)
    gidx = jnp.where(idx < L, b * L + idx, b * L + (ti & (L - 1)))
    gidx_ref[0] = gidx

    total = cum[L - 1:L, 0:1]
    offs = (lax.broadcasted_iota(jnp.int32, (1, 2 * NCHUNK), 1) * CH
            ).astype(jnp.float32)
    nr = jnp.clip(total - offs, 0.0, float(CH)).astype(jnp.int32)
    nreal_ref[0] = nr


def _tc_predict(x, w1t, b1r, g1r, be1r, w2t, b2r, g2r, be2r, wl, blr):
    full2 = lambda a: pl.BlockSpec(a, lambda b: (0, 0))
    full3 = lambda a: pl.BlockSpec(a, lambda b: (0, 0, 0))
    return pl.pallas_call(
        _tc_pred_body,
        grid=(B,),
        in_specs=[
            pl.BlockSpec((1, L, EMB), lambda b: (b, 0, 0)),
            full3((K, EMB, HID)), full2((1, HID)), full2((1, HID)), full2((1, HID)),
            full3((K, HID, HID)), full2((1, HID)), full2((1, HID)), full2((1, HID)),
            full2((HID, 1)), full2((1, 1)),
        ],
        out_specs=pl.BlockSpec((1, L, 1), lambda b: (b, 0, 0)),
        out_shape=jax.ShapeDtypeStruct((B, L, 1), jnp.float32),
    )(x, w1t, b1r, g1r, be1r, w2t, b2r, g2r, be2r, wl, blr)


def _tc_idx(t_col):
    return pl.pallas_call(
        _tc_idx_body,
        grid=(B,),
        in_specs=[pl.BlockSpec((1, L, 1), lambda b: (b, 0, 0))],
        out_specs=[
            pl.BlockSpec((1, 1, MEL), lambda b: (b, 0, 0)),
            pl.BlockSpec((1, 1, 2 * NCHUNK), lambda b: (b, 0, 0)),
        ],
        out_shape=[
            jax.ShapeDtypeStruct((B, 1, MEL), jnp.int32),
            jax.ShapeDtypeStruct((B, 1, 2 * NCHUNK), jnp.int32),
        ],
    )(t_col)


NBUF = 2


def _sc_expand_body(table_hbm, gidx_hbm, nr_hbm, out_hbm,
                    idx_all, *bufs_and_sems):
    rows_v = bufs_and_sems[:NBUF]
    nr_v = bufs_and_sems[NBUF]
    nr_sh = bufs_and_sems[NBUF + 1]
    sm = bufs_and_sems[NBUF + 2]
    gsems = bufs_and_sems[NBUF + 3:2 * NBUF + 3]
    ssems = bufs_and_sems[2 * NBUF + 3:3 * NBUF + 3]

    sid = lax.axis_index("s")
    wid = sid * NC + lax.axis_index("c")
    base = pl.multiple_of(wid * ROWS_PER_W, ROWS_PER_W)

    pltpu.sync_copy(gidx_hbm.at[wid], idx_all)
    pltpu.sync_copy(nr_hbm.at[wid], nr_v)
    pltpu.sync_copy(nr_v, nr_sh.at[sid])
    pltpu.sync_copy(nr_sh.at[sid], sm)
    nreal = [sm[c] for c in range(NCHUNK)]

    z16 = jnp.zeros((16,), jnp.float32)

    def start_gather(c):
        pltpu.async_copy(table_hbm.at[idx_all.at[c]], rows_v[c % NBUF],
                         gsems[c % NBUF])

    def wait_scat(c):
        bi = c % NBUF
        pltpu.make_async_copy(
            rows_v[bi], out_hbm.at[pl.ds(base + c * CH, CH)], ssems[bi]).wait()

    for c in range(min(NBUF, NCHUNK)):
        start_gather(c)

    for c in range(NCHUNK):
        bi = c % NBUF
        pltpu.make_async_copy(table_hbm.at[idx_all.at[c]], rows_v[bi],
                              gsems[bi]).wait()

        @pl.when(nreal[c] < CH)
        def _(c=c, bi=bi):
            rv = rows_v[bi]
            def zrow(r, _):
                for j in range(EMB // 16):
                    rv[r, pl.ds(j * 16, 16)] = z16
                return 0
            lax.fori_loop(nreal[c], CH, zrow, 0)

        pltpu.async_copy(rows_v[bi],
                         out_hbm.at[pl.ds(base + c * CH, CH)], ssems[bi])

        nc = c + NBUF
        if nc < NCHUNK:
            wait_scat(c)
            start_gather(nc)

    for c in range(max(NCHUNK - NBUF, 0), NCHUNK):
        wait_scat(c)


@functools.cache
def _sc_expand():
    return pl.kernel(
        _sc_expand_body,
        out_type=jax.ShapeDtypeStruct((ROWS, EMB), jnp.float32),
        mesh=plsc.VectorSubcoreMesh(core_axis_name="c", subcore_axis_name="s",
                                    num_cores=NC, num_subcores=NS),
        scratch_types=[
            pltpu.VMEM((NCHUNK, CH), jnp.int32),
            *[pltpu.VMEM((CH, EMB), jnp.float32) for _ in range(NBUF)],
            pltpu.VMEM((NCHUNK,), jnp.int32),
            pltpu.VMEM_SHARED((NS, NCHUNK), jnp.int32),
            pltpu.SMEM((NCHUNK,), jnp.int32),
            *[pltpu.SemaphoreType.DMA for _ in range(2 * NBUF)],
        ],
    )


def kernel(x, target, mel_max_len, W1, b1, g1, be1, W2, b2, g2, be2, Wl, bl):
    w1t = jnp.transpose(W1, (2, 1, 0))
    w2t = jnp.transpose(W2, (2, 1, 0))
    t_col = target.reshape(B, L, 1)
    row2 = lambda a: a.reshape(1, -1)

    gidx3, nreal3 = _tc_idx(t_col)
    out_flat = _sc_expand()(x.reshape(B * L, EMB),
                            gidx3.reshape(NW, NCHUNK, CH),
                            nreal3.reshape(NW, NCHUNK))
    dur = _tc_predict(
        x, w1t, row2(b1), row2(g1), row2(be1),
        w2t, row2(b2), row2(g2), row2(be2), Wl, bl.reshape(1, 1))
    output = out_flat.reshape(B, MEL, EMB)
    return (output, dur)
